# baseline (speedup 1.0000x reference)
import functools

import jax
import jax.numpy as jnp
from jax import lax
from jax.experimental import pallas as pl
from jax.experimental.pallas import tpu as pltpu
from jax.experimental.pallas import tpu_sc as plsc

L = 512
D = 256
H = 8
DH = D // H
VP = 64
V = 50
AV = 64
G = 8
NEG = -1e30
_dot1 = functools.partial(jnp.dot, preferred_element_type=jnp.float32)


def _split_bf(v):
    v_hi = v.astype(jnp.bfloat16)
    v_lo = (v - v_hi.astype(jnp.float32)).astype(jnp.bfloat16)
    return v_hi, v_lo


def _dot_rsplit(m_bf, v):
    v_hi, v_lo = _split_bf(v)
    return _dot1(m_bf, v_hi) + _dot1(m_bf, v_lo)


def _dot_lsplit(v, m_bf):
    v_hi, v_lo = _split_bf(v)
    return _dot1(v_hi, m_bf) + _dot1(v_lo, m_bf)


def _tc_body(se_ref, vx_ref, ttr_ref, admr_ref,
             wocc_ref, wvis_ref, wnxt_ref, a_ref,
             tv_ref, wcls_ref, bcls_ref, out_ref, log_ref):
    f32 = jnp.float32
    bf16 = jnp.bfloat16

    rows_a = lax.broadcasted_iota(jnp.int32, (AV, L), 0)
    lt_i = lax.broadcasted_iota(jnp.int32, (AV, AV), 0)
    lt_j = lax.broadcasted_iota(jnp.int32, (AV, AV), 1)
    ltri = (lt_j < lt_i).astype(bf16)
    rows_v = lax.broadcasted_iota(jnp.int32, (VP, L), 0).astype(f32)
    s_i = lax.broadcasted_iota(jnp.int32, (VP, VP), 0)
    s_j = lax.broadcasted_iota(jnp.int32, (VP, VP), 1)
    sub_diag = s_j == s_i - 1
    s_if = s_i.astype(f32)
    row_i = lax.broadcasted_iota(jnp.int32, (VP, D), 0).astype(f32)
    rt_bf = (lax.broadcasted_iota(jnp.int32, (D, H), 0) // DH
             == lax.broadcasted_iota(jnp.int32, (D, H), 1)).astype(bf16)
    rm_bf = (lax.broadcasted_iota(jnp.int32, (H, D), 1) // DH
             == lax.broadcasted_iota(jnp.int32, (H, D), 0)).astype(bf16)

    wocc_b = wocc_ref[...].astype(bf16)
    wvis_b = wvis_ref[...].astype(bf16)
    wnxt_b = wnxt_ref[...].astype(bf16)
    a0row = a_ref[0:1, :]
    a1row = a_ref[1:2, :]

    logit = _dot1(tv_ref[...].astype(bf16),
                  wcls_ref[...].astype(bf16)) + bcls_ref[...]

    for g in range(G):
        se = se_ref[g].astype(bf16)
        vx = vx_ref[g].astype(bf16)
        ttr = ttr_ref[g:g + 1, :]
        admr = admr_ref[g:g + 1, :]

        keep_r = (ttr != 5) & (admr != 0)
        occ_r = keep_r & (ttr == 1)

        oh_r = ((admr == rows_a) & keep_r).astype(f32)
        p_col = jnp.max(oh_r, axis=1, keepdims=True)
        rank_col = _dot1(ltri, p_col.astype(bf16))
        nv = jnp.sum(p_col)
        dst_r = jnp.sum(oh_r * rank_col, axis=0, keepdims=True)

        mf_b = (dst_r == rows_v) & occ_r
        mf = mf_b.astype(f32)
        mf_bf = mf_b.astype(bf16)
        mft = mf.T
        mft_b = mft > 0.5
        mft_bf = mft.astype(bf16)
        occ_cf = jnp.sum(mft, axis=1, keepdims=True)

        h_occ = _dot1(se, wocc_b)
        h_vis = _dot1(vx, wvis_b)
        h_nxt = _dot1(vx, wnxt_b)

        e_src = _dot_lsplit(h_occ * a0row, rt_bf)
        e_dst = _dot_lsplit(h_vis * a1row, rt_bf)
        ge = _dot_rsplit(mft_bf, e_dst)
        e = e_src + ge
        e = jnp.where(e > 0, e, 0.2 * e)

        e_tokmax = jnp.max(e, axis=1, keepdims=True)
        masked = jnp.where(mft_b, e_tokmax, NEG)
        m_row = jnp.max(masked, axis=0, keepdims=True)
        m_row = jnp.where(m_row > -1e29, m_row, 0.0)
        m_used = jnp.sum(mft * m_row, axis=1, keepdims=True)

        ex = jnp.exp(e - m_used) * occ_cf
        den = _dot_rsplit(mf_bf, ex)
        ex_rep = _dot_lsplit(ex, rm_bf)
        num = _dot_rsplit(mf_bf, ex_rep * h_occ)
        den_rep = _dot_lsplit(den, rm_bf)
        agg1 = num / jnp.maximum(den_rep, 1e-9)

        shift = (sub_diag & (s_if < nv)).astype(bf16)
        agg2 = _dot_rsplit(shift, h_nxt)

        pre = agg1 + agg2 + h_vis
        out = jnp.where(pre > 0, pre, jnp.exp(pre) - 1.0)
        out_ref[g] = jnp.where(row_i < nv, out, 0.0)[:V]
        log_ref[g] = logit


def _tc_call(se3, vx3, ttr, admr, w_occ, w_vis, w_next,
             a_row, tv, wcls, bcls, interpret=False):
    b = se3.shape[0]
    f32 = jnp.float32
    fixed = lambda *s: pl.BlockSpec(s, lambda i: (0,) * len(s))
    per_b = lambda *s: pl.BlockSpec(s, lambda i: (i,) + (0,) * (len(s) - 1))
    return pl.pallas_call(
        _tc_body,
        grid=(b // G,),
        in_specs=[
            per_b(G, L, D), per_b(G, VP, D),
            per_b(G, L), per_b(G, L),
            fixed(D, D), fixed(D, D), fixed(D, D),
            fixed(2, D),
            fixed(1, D), fixed(D, 1), fixed(1, 1),
        ],
        out_specs=[per_b(G, V, D), per_b(G, 1, 1)],
        out_shape=[
            jax.ShapeDtypeStruct((b, V, D), f32),
            jax.ShapeDtypeStruct((b, 1, 1), f32),
        ],
        compiler_params=pltpu.CompilerParams(
            dimension_semantics=("arbitrary",),
        ),
        interpret=interpret,
    )(se3, vx3, ttr, admr, w_occ, w_vis, w_next,
      a_row, tv, wcls, bcls)


def _sc_gather(table, idx_se, idx_vx=None):
    info = plsc.get_sparse_core_info()
    nw = info.num_cores * info.num_subcores
    n_se = idx_se.shape[0]
    n_vx = 0 if idx_vx is None else idx_vx.shape[0]
    ch = 128
    se_ch = n_se // (nw * ch)
    vx_ch = n_vx // (nw * ch)
    n_ch = se_ch + vx_ch
    per_w = n_ch * ch
    mesh = plsc.VectorSubcoreMesh(core_axis_name="c", subcore_axis_name="s")

    out_type = [jax.ShapeDtypeStruct((n_se, D), jnp.float32)]
    if idx_vx is not None:
        out_type.append(jax.ShapeDtypeStruct((n_vx, D), jnp.float32))

    @functools.partial(
        pl.kernel, mesh=mesh,
        out_type=out_type,
        scratch_types=[
            pltpu.VMEM((per_w,), jnp.int32),
            pltpu.VMEM((3, ch, D), jnp.float32),
            pltpu.SemaphoreType.DMA((3,)),
            pltpu.SemaphoreType.DMA((3,)),
        ],
    )
    def k(t_hbm, *args):
        if idx_vx is not None:
            ise_hbm, ivx_hbm, out_se, out_vx = args[:4]
            idx_v, rows_v, gsem, wsem = args[4:]
        else:
            ise_hbm, out_se = args[:2]
            idx_v, rows_v, gsem, wsem = args[2:]
        wid = lax.axis_index("s") * info.num_cores + lax.axis_index("c")
        base_se = wid * (se_ch * ch)
        base_vx = wid * (vx_ch * ch)
        pltpu.sync_copy(ise_hbm.at[pl.ds(base_se, se_ch * ch)],
                        idx_v.at[pl.ds(0, se_ch * ch)])
        if vx_ch:
            pltpu.sync_copy(ivx_hbm.at[pl.ds(base_vx, vx_ch * ch)],
                            idx_v.at[pl.ds(se_ch * ch, vx_ch * ch)])

        def _gather(c):
            return pltpu.make_async_copy(
                t_hbm.at[idx_v.at[pl.ds(c * ch, ch)]],
                rows_v.at[c % 3], gsem.at[c % 3])

        def _write(c):
            if c < se_ch:
                dst = out_se.at[pl.ds(base_se + c * ch, ch)]
            else:
                dst = out_vx.at[pl.ds(base_vx + (c - se_ch) * ch, ch)]
            return pltpu.make_async_copy(rows_v.at[c % 3], dst, wsem.at[c % 3])

        _gather(0).start()
        _gather(1).start()
        for c in range(n_ch):
            if c + 2 < n_ch:
                if c >= 1:
                    _write(c - 1).wait()
                _gather(c + 2).start()
            _gather(c).wait()
            _write(c).start()
        for c in range(max(0, n_ch - 3), n_ch):
            _write(c).wait()

    if idx_vx is not None:
        return k(table, idx_se, idx_vx)
    res = k(table, idx_se)
    return res[0] if isinstance(res, (list, tuple)) else res


def kernel(input_ids, token_types, adm_index, age_ids, diag_code_group_dicts,
           task_id, token_emb, task_emb_table, W_occ, W_vis, W_next,
           a_o2v, a_next, W_cls, b_cls):
    f32 = jnp.float32
    b = input_ids.shape[0]
    v = age_ids.shape[1]

    pad_idx = (jnp.arange(b * (VP - v), dtype=jnp.int32).reshape(b, VP - v)
               % token_emb.shape[0])
    age_pad = jnp.concatenate([age_ids.astype(jnp.int32), pad_idx], axis=1)
    ids = input_ids.astype(jnp.int32)
    hb = b // 2

    se_flat, vx_flat = _sc_gather(
        token_emb, ids.reshape(-1), age_pad.reshape(-1))
    vx3 = vx_flat.reshape(b, VP, D)

    ttr = token_types.astype(jnp.int32)
    admr = adm_index.astype(jnp.int32)
    a_row = a_o2v.reshape(2, D)
    tv = jnp.take(task_emb_table, jnp.asarray(task_id, jnp.int32),
                  axis=0).reshape(1, D)
    bcls = b_cls.reshape(1, 1)

    out_p, log3 = _tc_call(se_flat.reshape(b, L, D), vx3, ttr, admr,
                           W_occ, W_vis, W_next, a_row, tv, W_cls, bcls)
    return log3.reshape(b), out_p


# speedup vs baseline: 1.0312x; 1.0312x over previous
import functools

import jax
import jax.numpy as jnp
from jax import lax
from jax.experimental import pallas as pl
from jax.experimental.pallas import tpu as pltpu
from jax.experimental.pallas import tpu_sc as plsc

L = 512
D = 256
H = 8
DH = D // H
VP = 64
V = 50
AV = 64
G = 8
NEG = -1e30
_dot1 = functools.partial(jnp.dot, preferred_element_type=jnp.float32)


def _split_bf(v):
    v_hi = v.astype(jnp.bfloat16)
    v_lo = (v - v_hi.astype(jnp.float32)).astype(jnp.bfloat16)
    return v_hi, v_lo


def _dot_rsplit(m_bf, v):
    v_hi, v_lo = _split_bf(v)
    return _dot1(m_bf, v_hi) + _dot1(m_bf, v_lo)


def _dot_lsplit(v, m_bf):
    v_hi, v_lo = _split_bf(v)
    return _dot1(v_hi, m_bf) + _dot1(v_lo, m_bf)


def _tc_body(se_ref, vx_ref, ttr_ref, admr_ref,
             wocc_ref, wvis_ref, wnxt_ref, a_ref,
             tv_ref, wcls_ref, bcls_ref, out_ref, log_ref):
    f32 = jnp.float32
    bf16 = jnp.bfloat16

    rows_a = lax.broadcasted_iota(jnp.int32, (AV, L), 0)
    lt_i = lax.broadcasted_iota(jnp.int32, (AV, AV), 0)
    lt_j = lax.broadcasted_iota(jnp.int32, (AV, AV), 1)
    ltri = (lt_j < lt_i).astype(bf16)
    rows_v = lax.broadcasted_iota(jnp.int32, (VP, L), 0).astype(f32)
    s_i = lax.broadcasted_iota(jnp.int32, (VP, VP), 0)
    s_j = lax.broadcasted_iota(jnp.int32, (VP, VP), 1)
    sub_diag = s_j == s_i - 1
    s_if = s_i.astype(f32)
    row_i = lax.broadcasted_iota(jnp.int32, (VP, D), 0).astype(f32)
    rt_bf = (lax.broadcasted_iota(jnp.int32, (D, H), 0) // DH
             == lax.broadcasted_iota(jnp.int32, (D, H), 1)).astype(bf16)
    rm_bf = (lax.broadcasted_iota(jnp.int32, (H, D), 1) // DH
             == lax.broadcasted_iota(jnp.int32, (H, D), 0)).astype(bf16)

    wocc_b = wocc_ref[...].astype(bf16)
    wvis_b = wvis_ref[...].astype(bf16)
    wnxt_b = wnxt_ref[...].astype(bf16)
    a0row = a_ref[0:1, :]
    a1row = a_ref[1:2, :]

    logit = _dot1(tv_ref[...].astype(bf16),
                  wcls_ref[...].astype(bf16)) + bcls_ref[...]

    for g in range(G):
        se = se_ref[g].astype(bf16)
        vx = vx_ref[g].astype(bf16)
        ttr = ttr_ref[g:g + 1, :]
        admr = admr_ref[g:g + 1, :]

        keep_r = (ttr != 5) & (admr != 0)
        occ_r = keep_r & (ttr == 1)

        oh_r = ((admr == rows_a) & keep_r).astype(f32)
        p_col = jnp.max(oh_r, axis=1, keepdims=True)
        rank_col = _dot1(ltri, p_col.astype(bf16))
        nv = jnp.sum(p_col)
        dst_r = jnp.sum(oh_r * rank_col, axis=0, keepdims=True)

        mf_b = (dst_r == rows_v) & occ_r
        mf = mf_b.astype(f32)
        mf_bf = mf_b.astype(bf16)
        mft = mf.T
        mft_b = mft > 0.5
        mft_bf = mft.astype(bf16)
        occ_cf = jnp.sum(mft, axis=1, keepdims=True)

        h_occ = _dot1(se, wocc_b)
        h_vis = _dot1(vx, wvis_b)
        h_nxt = _dot1(vx, wnxt_b)

        e_src = _dot_lsplit(h_occ * a0row, rt_bf)
        e_dst = _dot_lsplit(h_vis * a1row, rt_bf)
        ge = _dot_rsplit(mft_bf, e_dst)
        e = e_src + ge
        e = jnp.where(e > 0, e, 0.2 * e)

        e_tokmax = jnp.max(e, axis=1, keepdims=True)
        masked = jnp.where(mft_b, e_tokmax, NEG)
        m_row = jnp.max(masked, axis=0, keepdims=True)
        m_row = jnp.where(m_row > -1e29, m_row, 0.0)
        m_used = jnp.sum(mft * m_row, axis=1, keepdims=True)

        ex = jnp.exp(e - m_used) * occ_cf
        den = _dot_rsplit(mf_bf, ex)
        ex_rep = _dot_lsplit(ex, rm_bf)
        num = _dot_rsplit(mf_bf, ex_rep * h_occ)
        den_rep = _dot_lsplit(den, rm_bf)
        agg1 = num / jnp.maximum(den_rep, 1e-9)

        shift = (sub_diag & (s_if < nv)).astype(bf16)
        agg2 = _dot_rsplit(shift, h_nxt)

        pre = agg1 + agg2 + h_vis
        out = jnp.where(pre > 0, pre, jnp.exp(pre) - 1.0)
        out_ref[g] = jnp.where(row_i < nv, out, 0.0)[:V]
        log_ref[g] = logit


def _tc_call(se3, vx3, ttr, admr, w_occ, w_vis, w_next,
             a_row, tv, wcls, bcls):
    b = se3.shape[0]
    f32 = jnp.float32
    fixed = lambda *s: pl.BlockSpec(s, lambda i: (0,) * len(s))
    per_b = lambda *s: pl.BlockSpec(s, lambda i: (i,) + (0,) * (len(s) - 1))
    return pl.pallas_call(
        _tc_body,
        grid=(b // G,),
        in_specs=[
            per_b(G, L, D), per_b(G, VP, D),
            per_b(G, L), per_b(G, L),
            fixed(D, D), fixed(D, D), fixed(D, D),
            fixed(2, D),
            fixed(1, D), fixed(D, 1), fixed(1, 1),
        ],
        out_specs=[per_b(G, V, D), per_b(G, 1, 1)],
        out_shape=[
            jax.ShapeDtypeStruct((b, V, D), f32),
            jax.ShapeDtypeStruct((b, 1, 1), f32),
        ],
        compiler_params=pltpu.CompilerParams(
            dimension_semantics=("arbitrary",),
        ),
    )(se3, vx3, ttr, admr, w_occ, w_vis, w_next,
      a_row, tv, wcls, bcls)


def _sc_gather(table, idx_se, idx_vx=None):
    info = plsc.get_sparse_core_info()
    nw = info.num_cores * info.num_subcores
    n_se = idx_se.shape[0]
    n_vx = 0 if idx_vx is None else idx_vx.shape[0]
    ch = 128
    se_ch = n_se // (nw * ch)
    vx_ch = n_vx // (nw * ch)
    n_ch = se_ch + vx_ch
    per_w = n_ch * ch
    mesh = plsc.VectorSubcoreMesh(core_axis_name="c", subcore_axis_name="s")

    out_type = [jax.ShapeDtypeStruct((n_se, D), jnp.float32)]
    if idx_vx is not None:
        out_type.append(jax.ShapeDtypeStruct((n_vx, D), jnp.float32))

    @functools.partial(
        pl.kernel, mesh=mesh,
        out_type=out_type,
        scratch_types=[
            pltpu.VMEM((per_w,), jnp.int32),
            pltpu.VMEM((3, ch, D), jnp.float32),
            pltpu.SemaphoreType.DMA((3,)),
            pltpu.SemaphoreType.DMA((3,)),
        ],
    )
    def k(t_hbm, *args):
        if idx_vx is not None:
            ise_hbm, ivx_hbm, out_se, out_vx = args[:4]
            idx_v, rows_v, gsem, wsem = args[4:]
        else:
            ise_hbm, out_se = args[:2]
            idx_v, rows_v, gsem, wsem = args[2:]
        wid = lax.axis_index("s") * info.num_cores + lax.axis_index("c")
        base_se = wid * (se_ch * ch)
        base_vx = wid * (vx_ch * ch)
        pltpu.sync_copy(ise_hbm.at[pl.ds(base_se, se_ch * ch)],
                        idx_v.at[pl.ds(0, se_ch * ch)])
        if vx_ch:
            pltpu.sync_copy(ivx_hbm.at[pl.ds(base_vx, vx_ch * ch)],
                            idx_v.at[pl.ds(se_ch * ch, vx_ch * ch)])

        def _gather(c):
            return pltpu.make_async_copy(
                t_hbm.at[idx_v.at[pl.ds(c * ch, ch)]],
                rows_v.at[c % 3], gsem.at[c % 3])

        def _write(c):
            if c < se_ch:
                dst = out_se.at[pl.ds(base_se + c * ch, ch)]
            else:
                dst = out_vx.at[pl.ds(base_vx + (c - se_ch) * ch, ch)]
            return pltpu.make_async_copy(rows_v.at[c % 3], dst, wsem.at[c % 3])

        _gather(0).start()
        _gather(1).start()
        for c in range(n_ch):
            if c + 2 < n_ch:
                if c >= 1:
                    _write(c - 1).wait()
                _gather(c + 2).start()
            _gather(c).wait()
            _write(c).start()
        for c in range(max(0, n_ch - 3), n_ch):
            _write(c).wait()

    if idx_vx is not None:
        return k(table, idx_se, idx_vx)
    res = k(table, idx_se)
    return res[0] if isinstance(res, (list, tuple)) else res


def kernel(input_ids, token_types, adm_index, age_ids, diag_code_group_dicts,
           task_id, token_emb, task_emb_table, W_occ, W_vis, W_next,
           a_o2v, a_next, W_cls, b_cls):
    f32 = jnp.float32
    b = input_ids.shape[0]
    v = age_ids.shape[1]

    pad_idx = (jnp.arange(b * (VP - v), dtype=jnp.int32).reshape(b, VP - v)
               % token_emb.shape[0])
    age_pad = jnp.concatenate([age_ids.astype(jnp.int32), pad_idx], axis=1)
    ids = input_ids.astype(jnp.int32)
    hb = b // 2

    se1_flat = _sc_gather(token_emb, ids[:hb].reshape(-1))
    se2_flat, vx_flat = _sc_gather(
        token_emb, ids[hb:].reshape(-1), age_pad.reshape(-1))
    vx3 = vx_flat.reshape(b, VP, D)

    ttr = token_types.astype(jnp.int32)
    admr = adm_index.astype(jnp.int32)
    a_row = a_o2v.reshape(2, D)
    tv = jnp.take(task_emb_table, jnp.asarray(task_id, jnp.int32),
                  axis=0).reshape(1, D)
    bcls = b_cls.reshape(1, 1)

    out1, log1 = _tc_call(se1_flat.reshape(hb, L, D), vx3[:hb],
                          ttr[:hb], admr[:hb],
                          W_occ, W_vis, W_next, a_row, tv, W_cls, bcls)
    out2, log2 = _tc_call(se2_flat.reshape(hb, L, D), vx3[hb:],
                          ttr[hb:], admr[hb:],
                          W_occ, W_vis, W_next, a_row, tv, W_cls, bcls)
    out_p = jnp.concatenate([out1, out2], axis=0)
    log3 = jnp.concatenate([log1, log2], axis=0)
    return log3.reshape(b), out_p
